# baseline (speedup 1.0000x reference)
import jax
import jax.numpy as jnp
from jax import lax
from jax.experimental import pallas as pl
from jax.experimental.pallas import tpu as pltpu
from jax.experimental.pallas import tpu_sc as plsc

N_OBS = 1_000_000
NUM_TAB = 4096

NW = 32
B = 1600
NBLK = N_OBS // B
CHUNKS = B // 16
PW = 8


def _rot_conj(qx, qy, qz, qw, vx, vy, vz):
    tx = -2.0 * (qy * vz - qz * vy)
    ty = -2.0 * (qz * vx - qx * vz)
    tz = -2.0 * (qx * vy - qy * vx)
    ox = vx + qw * tx - (qy * tz - qz * ty)
    oy = vy + qw * ty - (qz * tx - qx * tz)
    oz = vz + qw * tz - (qx * ty - qy * tx)
    return ox, oy, oz


def _body(nblk, gidx_hbm, midx_hbm, pidx_hbm, fx_hbm, fy_hbm, fz_hbm,
          gtab_hbm, mtab_hbm, pts_hbm,
          ox_hbm, oy_hbm, oz_hbm,
          gtab_v, mtab_v,
          gidx0, midx0, pidx0, fx0, fy0, fz0, pts0, ox0, oy0, oz0,
          gidx1, midx1, pidx1, fx1, fy1, fz1, pts1, ox1, oy1, oz1,
          semA0, semA1, semG0, semG1, semO0, semO1):
    cid = lax.axis_index("c")
    sid = lax.axis_index("s")
    wid = sid * 2 + cid

    pltpu.sync_copy(gtab_hbm, gtab_v)
    pltpu.sync_copy(mtab_hbm, mtab_v)

    col = [jnp.full((16,), c, jnp.int32) for c in range(8)]
    nb = (nblk - wid + NW - 1) // NW

    s0 = (gidx0, midx0, pidx0, fx0, fy0, fz0, pts0, ox0, oy0, oz0,
          semA0, semG0, semO0)
    s1 = (gidx1, midx1, pidx1, fx1, fy1, fz1, pts1, ox1, oy1, oz1,
          semA1, semG1, semO1)

    def streams(s):
        return ((gidx_hbm, s[0]), (midx_hbm, s[1]), (pidx_hbm, s[2]),
                (fx_hbm, s[3]), (fy_hbm, s[4]), (fz_hbm, s[5]))

    def issue_lin(k, s):
        base = (wid + k * NW) * B
        for hbm, buf in streams(s):
            pltpu.async_copy(hbm.at[pl.ds(base, B)], buf, s[10])

    def drain_lin(s):
        for hbm, buf in streams(s):
            pltpu.make_async_copy(hbm.at[pl.ds(0, B)], buf, s[10]).wait()

    def issue_gather(s):
        pltpu.async_copy(pts_hbm.at[s[2]], s[6], s[11])

    def drain_gather(s):
        pltpu.make_async_copy(pts_hbm.at[pl.ds(0, B)], s[6], s[11]).wait()

    def issue_out(k, s):
        base = (wid + k * NW) * B
        pltpu.async_copy(s[7], ox_hbm.at[pl.ds(base, B)], s[12])
        pltpu.async_copy(s[8], oy_hbm.at[pl.ds(base, B)], s[12])
        pltpu.async_copy(s[9], oz_hbm.at[pl.ds(base, B)], s[12])

    def drain_out(s):
        pltpu.make_async_copy(s[7], ox_hbm.at[pl.ds(0, B)], s[12]).wait()
        pltpu.make_async_copy(s[8], oy_hbm.at[pl.ds(0, B)], s[12]).wait()
        pltpu.make_async_copy(s[9], oz_hbm.at[pl.ds(0, B)], s[12]).wait()

    def compute(s):
        gidx_v, midx_v, _, fx_v, fy_v, fz_v, pts_v, ox_v, oy_v, oz_v = s[:10]

        def chunk_body(k, carry2):
            o = pl.multiple_of(k * 16, 16)
            obs = k * 16 + lax.iota(jnp.int32, 16)
            gi = gidx_v[pl.ds(o, 16)]
            mi = midx_v[pl.ds(o, 16)]

            def gcol(c):
                return plsc.load_gather(gtab_v, [col[c], gi])

            def mcol(c):
                return plsc.load_gather(mtab_v, [col[c], mi])

            rqx, rqy, rqz, rqw = gcol(0), gcol(1), gcol(2), gcol(3)
            rtx, rty, rtz, fac = gcol(4), gcol(5), gcol(6), gcol(7)
            mqx, mqy, mqz, mqw = mcol(0), mcol(1), mcol(2), mcol(3)
            mtx, mty, mtz = mcol(4), mcol(5), mcol(6)

            fx = fx_v[pl.ds(o, 16)]
            fy = fy_v[pl.ds(o, 16)]
            fz = fz_v[pl.ds(o, 16)]
            px = plsc.load_gather(pts_v, [obs, col[0]])
            py = plsc.load_gather(pts_v, [obs, col[1]])
            pz = plsc.load_gather(pts_v, [obs, col[2]])

            ux = mtx - fx
            uy = mty - fy
            uz = mtz - fz
            vx, vy, vz = _rot_conj(mqx, mqy, mqz, mqw, ux, uy, uz)
            wx = vx + rtx
            wy = vy + rty
            wz = vz + rtz
            xx, xy, xz = _rot_conj(rqx, rqy, rqz, rqw, wx, wy, wz)

            ox_v[pl.ds(o, 16)] = fac * (px + xx)
            oy_v[pl.ds(o, 16)] = fac * (py + xy)
            oz_v[pl.ds(o, 16)] = fac * (pz + xz)
            return carry2

        lax.fori_loop(0, CHUNKS, chunk_body, 0)

    issue_lin(0, s0)
    drain_lin(s0)
    issue_gather(s0)
    issue_lin(1, s1)

    def body(j, carry):
        k0 = 2 * j
        k1 = 2 * j + 1

        @pl.when(k1 < nb)
        def _():
            drain_lin(s1)
            issue_gather(s1)

        @pl.when(j > 0)
        def _():
            drain_out(s0)

        drain_gather(s0)
        compute(s0)
        issue_out(k0, s0)

        @pl.when(k0 + 2 < nb)
        def _():
            issue_lin(k0 + 2, s0)

        @pl.when(k1 < nb)
        def _():
            @pl.when(j > 0)
            def _():
                drain_out(s1)

            drain_gather(s1)
            compute(s1)
            issue_out(k1, s1)

        @pl.when(k0 + 2 < nb)
        def _():
            drain_lin(s0)
            issue_gather(s0)

        @pl.when(k1 + 2 < nb)
        def _():
            issue_lin(k1 + 2, s1)

        return carry

    lax.fori_loop(0, (nb + 1) // 2, body, 0)
    drain_out(s0)

    @pl.when(nb >= 2)
    def _():
        drain_out(s1)


def _sc_call(gidx, midx, pidx, fx, fy, fz, gtab, mtab, pts):
    n = gidx.shape[0]
    nblk = n // B
    mesh = plsc.VectorSubcoreMesh(core_axis_name="c", subcore_axis_name="s")
    slot = [
        pltpu.VMEM((B,), jnp.int32),
        pltpu.VMEM((B,), jnp.int32),
        pltpu.VMEM((B,), jnp.int32),
        pltpu.VMEM((B,), jnp.float32),
        pltpu.VMEM((B,), jnp.float32),
        pltpu.VMEM((B,), jnp.float32),
        pltpu.VMEM((B, PW), jnp.float32),
        pltpu.VMEM((B,), jnp.float32),
        pltpu.VMEM((B,), jnp.float32),
        pltpu.VMEM((B,), jnp.float32),
    ]
    import functools
    f = pl.kernel(
        functools.partial(_body, nblk),
        out_type=(jax.ShapeDtypeStruct((n,), jnp.float32),
                  jax.ShapeDtypeStruct((n,), jnp.float32),
                  jax.ShapeDtypeStruct((n,), jnp.float32)),
        mesh=mesh,
        scratch_types=[
            pltpu.VMEM((8, NUM_TAB), jnp.float32),
            pltpu.VMEM((7, NUM_TAB), jnp.float32),
        ] + slot + slot + [
            pltpu.SemaphoreType.DMA,
            pltpu.SemaphoreType.DMA,
            pltpu.SemaphoreType.DMA,
            pltpu.SemaphoreType.DMA,
            pltpu.SemaphoreType.DMA,
            pltpu.SemaphoreType.DMA,
        ],
        compiler_params=pltpu.CompilerParams(
            use_tc_tiling_on_sc=False, needs_layout_passes=False),
    )
    return f(gidx, midx, pidx, fx, fy, fz, gtab, mtab, pts)


SPANS = ((0, 256000), (256000, 512000), (512000, 768000), (768000, N_OBS))


def kernel(feature_undist, grouping_indices, point_indices, is_calibrated,
           ref_rots, rel_rots, rel_trans, points_3d, scales, ref_trans):
    fac = jnp.where(is_calibrated, 1.0, 0.5).astype(jnp.float32)
    gtab = jnp.concatenate([ref_rots.T, ref_trans.T, fac[None, :]], axis=0)
    mtab = jnp.concatenate([rel_rots.T, rel_trans.T], axis=0)
    pts = jnp.pad(points_3d, ((0, 0), (0, PW - 3)))
    sf = feature_undist * scales
    outs = []
    for lo, hi in SPANS:
        ox, oy, oz = _sc_call(
            grouping_indices[lo:hi, 0].astype(jnp.int32),
            grouping_indices[lo:hi, 1].astype(jnp.int32),
            point_indices[lo:hi].astype(jnp.int32),
            sf[lo:hi, 0], sf[lo:hi, 1], sf[lo:hi, 2], gtab, mtab, pts)
        outs.append(jnp.stack([ox, oy, oz], axis=-1))
    return jnp.concatenate(outs, axis=0)


# speedup vs baseline: 1.1305x; 1.1305x over previous
import jax
import jax.numpy as jnp
from jax import lax
from jax.experimental import pallas as pl
from jax.experimental.pallas import tpu as pltpu
from jax.experimental.pallas import tpu_sc as plsc

N_OBS = 1_000_000
NUM_TAB = 4096

NW = 32
B = 1600
NBLK = N_OBS // B
CHUNKS = B // 16
PW = 8


def _rot_conj(qx, qy, qz, qw, vx, vy, vz):
    tx = -2.0 * (qy * vz - qz * vy)
    ty = -2.0 * (qz * vx - qx * vz)
    tz = -2.0 * (qx * vy - qy * vx)
    ox = vx + qw * tx - (qy * tz - qz * ty)
    oy = vy + qw * ty - (qz * tx - qx * tz)
    oz = vz + qw * tz - (qx * ty - qy * tx)
    return ox, oy, oz


def _body(nblk, gidx_hbm, midx_hbm, pidx_hbm, fx_hbm, fy_hbm, fz_hbm,
          gtab_hbm, mtab_hbm, pts_hbm,
          ox_hbm, oy_hbm, oz_hbm,
          gtab_v, mtab_v,
          gidx0, midx0, pidx0, fx0, fy0, fz0, pts0, ox0, oy0, oz0,
          gidx1, midx1, pidx1, fx1, fy1, fz1, pts1, ox1, oy1, oz1,
          semA0, semA1, semG0, semG1, semO0, semO1):
    cid = lax.axis_index("c")
    sid = lax.axis_index("s")
    wid = sid * 2 + cid

    pltpu.sync_copy(gtab_hbm, gtab_v)
    pltpu.sync_copy(mtab_hbm, mtab_v)

    col = [jnp.full((16,), c, jnp.int32) for c in range(8)]
    nb = (nblk - wid + NW - 1) // NW

    s0 = (gidx0, midx0, pidx0, fx0, fy0, fz0, pts0, ox0, oy0, oz0,
          semA0, semG0, semO0)
    s1 = (gidx1, midx1, pidx1, fx1, fy1, fz1, pts1, ox1, oy1, oz1,
          semA1, semG1, semO1)

    def streams(s):
        return ((gidx_hbm, s[0]), (midx_hbm, s[1]), (pidx_hbm, s[2]),
                (fx_hbm, s[3]), (fy_hbm, s[4]), (fz_hbm, s[5]))

    def issue_lin(k, s):
        base = (wid + k * NW) * B
        for hbm, buf in streams(s):
            pltpu.async_copy(hbm.at[pl.ds(base, B)], buf, s[10])

    def drain_lin(s):
        for hbm, buf in streams(s):
            pltpu.make_async_copy(hbm.at[pl.ds(0, B)], buf, s[10]).wait()

    def issue_gather(s):
        pltpu.async_copy(pts_hbm.at[s[2]], s[6], s[11])

    def drain_gather(s):
        pltpu.make_async_copy(pts_hbm.at[pl.ds(0, B)], s[6], s[11]).wait()

    def issue_out(k, s):
        base = (wid + k * NW) * B
        pltpu.async_copy(s[7], ox_hbm.at[pl.ds(base, B)], s[12])
        pltpu.async_copy(s[8], oy_hbm.at[pl.ds(base, B)], s[12])
        pltpu.async_copy(s[9], oz_hbm.at[pl.ds(base, B)], s[12])

    def drain_out(s):
        pltpu.make_async_copy(s[7], ox_hbm.at[pl.ds(0, B)], s[12]).wait()
        pltpu.make_async_copy(s[8], oy_hbm.at[pl.ds(0, B)], s[12]).wait()
        pltpu.make_async_copy(s[9], oz_hbm.at[pl.ds(0, B)], s[12]).wait()

    def compute(s):
        gidx_v, midx_v, _, fx_v, fy_v, fz_v, pts_v, ox_v, oy_v, oz_v = s[:10]

        def chunk_body(k, carry2):
            o = pl.multiple_of(k * 16, 16)
            obs = k * 16 + lax.iota(jnp.int32, 16)
            gi = gidx_v[pl.ds(o, 16)]
            mi = midx_v[pl.ds(o, 16)]

            def gcol(c):
                return plsc.load_gather(gtab_v, [col[c], gi])

            def mcol(c):
                return plsc.load_gather(mtab_v, [col[c], mi])

            rqx, rqy, rqz, rqw = gcol(0), gcol(1), gcol(2), gcol(3)
            rtx, rty, rtz, fac = gcol(4), gcol(5), gcol(6), gcol(7)
            mqx, mqy, mqz, mqw = mcol(0), mcol(1), mcol(2), mcol(3)
            mtx, mty, mtz = mcol(4), mcol(5), mcol(6)

            fx = fx_v[pl.ds(o, 16)]
            fy = fy_v[pl.ds(o, 16)]
            fz = fz_v[pl.ds(o, 16)]
            px = plsc.load_gather(pts_v, [obs, col[0]])
            py = plsc.load_gather(pts_v, [obs, col[1]])
            pz = plsc.load_gather(pts_v, [obs, col[2]])

            ux = mtx - fx
            uy = mty - fy
            uz = mtz - fz
            vx, vy, vz = _rot_conj(mqx, mqy, mqz, mqw, ux, uy, uz)
            wx = vx + rtx
            wy = vy + rty
            wz = vz + rtz
            xx, xy, xz = _rot_conj(rqx, rqy, rqz, rqw, wx, wy, wz)

            ox_v[pl.ds(o, 16)] = fac * (px + xx)
            oy_v[pl.ds(o, 16)] = fac * (py + xy)
            oz_v[pl.ds(o, 16)] = fac * (pz + xz)
            return carry2

        lax.fori_loop(0, CHUNKS, chunk_body, 0)

    issue_lin(0, s0)
    drain_lin(s0)
    issue_gather(s0)
    issue_lin(1, s1)

    def body(j, carry):
        k0 = 2 * j
        k1 = 2 * j + 1

        @pl.when(k1 < nb)
        def _():
            drain_lin(s1)
            issue_gather(s1)

        @pl.when(j > 0)
        def _():
            drain_out(s0)

        drain_gather(s0)
        compute(s0)
        issue_out(k0, s0)

        @pl.when(k0 + 2 < nb)
        def _():
            issue_lin(k0 + 2, s0)

        @pl.when(k1 < nb)
        def _():
            @pl.when(j > 0)
            def _():
                drain_out(s1)

            drain_gather(s1)
            compute(s1)
            issue_out(k1, s1)

        @pl.when(k0 + 2 < nb)
        def _():
            drain_lin(s0)
            issue_gather(s0)

        @pl.when(k1 + 2 < nb)
        def _():
            issue_lin(k1 + 2, s1)

        return carry

    lax.fori_loop(0, (nb + 1) // 2, body, 0)
    drain_out(s0)

    @pl.when(nb >= 2)
    def _():
        drain_out(s1)


def _sc_call(gidx, midx, pidx, fx, fy, fz, gtab, mtab, pts):
    n = gidx.shape[0]
    nblk = n // B
    mesh = plsc.VectorSubcoreMesh(core_axis_name="c", subcore_axis_name="s")
    slot = [
        pltpu.VMEM((B,), jnp.int32),
        pltpu.VMEM((B,), jnp.int32),
        pltpu.VMEM((B,), jnp.int32),
        pltpu.VMEM((B,), jnp.float32),
        pltpu.VMEM((B,), jnp.float32),
        pltpu.VMEM((B,), jnp.float32),
        pltpu.VMEM((B, PW), jnp.float32),
        pltpu.VMEM((B,), jnp.float32),
        pltpu.VMEM((B,), jnp.float32),
        pltpu.VMEM((B,), jnp.float32),
    ]
    import functools
    f = pl.kernel(
        functools.partial(_body, nblk),
        out_type=(jax.ShapeDtypeStruct((n,), jnp.float32),
                  jax.ShapeDtypeStruct((n,), jnp.float32),
                  jax.ShapeDtypeStruct((n,), jnp.float32)),
        mesh=mesh,
        scratch_types=[
            pltpu.VMEM((8, NUM_TAB), jnp.float32),
            pltpu.VMEM((7, NUM_TAB), jnp.float32),
        ] + slot + slot + [
            pltpu.SemaphoreType.DMA,
            pltpu.SemaphoreType.DMA,
            pltpu.SemaphoreType.DMA,
            pltpu.SemaphoreType.DMA,
            pltpu.SemaphoreType.DMA,
            pltpu.SemaphoreType.DMA,
        ],
        compiler_params=pltpu.CompilerParams(
            use_tc_tiling_on_sc=False, needs_layout_passes=False),
    )
    return f(gidx, midx, pidx, fx, fy, fz, gtab, mtab, pts)


SPANS = ((0, 512000), (512000, N_OBS))


def kernel(feature_undist, grouping_indices, point_indices, is_calibrated,
           ref_rots, rel_rots, rel_trans, points_3d, scales, ref_trans):
    fac = jnp.where(is_calibrated, 1.0, 0.5).astype(jnp.float32)
    gtab = jnp.concatenate([ref_rots.T, ref_trans.T, fac[None, :]], axis=0)
    mtab = jnp.concatenate([rel_rots.T, rel_trans.T], axis=0)
    pts = jnp.pad(points_3d, ((0, 0), (0, PW - 3)))
    sf = feature_undist * scales
    outs = []
    for lo, hi in SPANS:
        ox, oy, oz = _sc_call(
            grouping_indices[lo:hi, 0].astype(jnp.int32),
            grouping_indices[lo:hi, 1].astype(jnp.int32),
            point_indices[lo:hi].astype(jnp.int32),
            sf[lo:hi, 0], sf[lo:hi, 1], sf[lo:hi, 2], gtab, mtab, pts)
        outs.append(jnp.stack([ox, oy, oz], axis=-1))
    return jnp.concatenate(outs, axis=0)


# speedup vs baseline: 1.2532x; 1.1086x over previous
import jax
import jax.numpy as jnp
from jax import lax
from jax.experimental import pallas as pl
from jax.experimental.pallas import tpu as pltpu
from jax.experimental.pallas import tpu_sc as plsc

N_OBS = 1_000_000
NUM_TAB = 4096

NW = 32
B = 1600
NBLK = N_OBS // B
CHUNKS = B // 16
PW = 8


def _rot_conj(qx, qy, qz, qw, vx, vy, vz):
    tx = -2.0 * (qy * vz - qz * vy)
    ty = -2.0 * (qz * vx - qx * vz)
    tz = -2.0 * (qx * vy - qy * vx)
    ox = vx + qw * tx - (qy * tz - qz * ty)
    oy = vy + qw * ty - (qz * tx - qx * tz)
    oz = vz + qw * tz - (qx * ty - qy * tx)
    return ox, oy, oz


def _body(nblk, gidx_hbm, midx_hbm, pidx_hbm, fx_hbm, fy_hbm, fz_hbm,
          scal_hbm,
          gtab_hbm, mtab_hbm, pts_hbm,
          ox_hbm, oy_hbm, oz_hbm,
          gtab_v, mtab_v,
          gidx0, midx0, pidx0, fx0, fy0, fz0, scal0, pts0, ox0, oy0, oz0,
          gidx1, midx1, pidx1, fx1, fy1, fz1, scal1, pts1, ox1, oy1, oz1,
          semA0, semA1, semG0, semG1, semO0, semO1):
    cid = lax.axis_index("c")
    sid = lax.axis_index("s")
    wid = sid * 2 + cid

    pltpu.sync_copy(gtab_hbm, gtab_v)
    pltpu.sync_copy(mtab_hbm, mtab_v)

    col = [jnp.full((16,), c, jnp.int32) for c in range(8)]
    nb = (nblk - wid + NW - 1) // NW

    s0 = (gidx0, midx0, pidx0, fx0, fy0, fz0, scal0, pts0, ox0, oy0, oz0,
          semA0, semG0, semO0)
    s1 = (gidx1, midx1, pidx1, fx1, fy1, fz1, scal1, pts1, ox1, oy1, oz1,
          semA1, semG1, semO1)

    def streams(s):
        return ((gidx_hbm, s[0]), (midx_hbm, s[1]), (pidx_hbm, s[2]),
                (fx_hbm, s[3]), (fy_hbm, s[4]), (fz_hbm, s[5]),
                (scal_hbm, s[6]))

    def issue_lin(k, s):
        base = (wid + k * NW) * B
        for hbm, buf in streams(s):
            pltpu.async_copy(hbm.at[pl.ds(base, B)], buf, s[11])

    def drain_lin(s):
        for hbm, buf in streams(s):
            pltpu.make_async_copy(hbm.at[pl.ds(0, B)], buf, s[11]).wait()

    def issue_gather(s):
        pltpu.async_copy(pts_hbm.at[s[2]], s[7], s[12])

    def drain_gather(s):
        pltpu.make_async_copy(pts_hbm.at[pl.ds(0, B)], s[7], s[12]).wait()

    def issue_out(k, s):
        base = (wid + k * NW) * B
        pltpu.async_copy(s[8], ox_hbm.at[pl.ds(base, B)], s[13])
        pltpu.async_copy(s[9], oy_hbm.at[pl.ds(base, B)], s[13])
        pltpu.async_copy(s[10], oz_hbm.at[pl.ds(base, B)], s[13])

    def drain_out(s):
        pltpu.make_async_copy(s[8], ox_hbm.at[pl.ds(0, B)], s[13]).wait()
        pltpu.make_async_copy(s[9], oy_hbm.at[pl.ds(0, B)], s[13]).wait()
        pltpu.make_async_copy(s[10], oz_hbm.at[pl.ds(0, B)], s[13]).wait()

    def compute(s):
        gidx_v, midx_v, _, fx_v, fy_v, fz_v, scal_v, pts_v, ox_v, oy_v, oz_v \
            = s[:11]

        def chunk_body(k, carry2):
            o = pl.multiple_of(k * 16, 16)
            obs = k * 16 + lax.iota(jnp.int32, 16)
            gi = gidx_v[pl.ds(o, 16)]
            mi = midx_v[pl.ds(o, 16)]

            def gcol(c):
                return plsc.load_gather(gtab_v, [col[c], gi])

            def mcol(c):
                return plsc.load_gather(mtab_v, [col[c], mi])

            rqx, rqy, rqz, rqw = gcol(0), gcol(1), gcol(2), gcol(3)
            rtx, rty, rtz, fac = gcol(4), gcol(5), gcol(6), gcol(7)
            mqx, mqy, mqz, mqw = mcol(0), mcol(1), mcol(2), mcol(3)
            mtx, mty, mtz = mcol(4), mcol(5), mcol(6)

            fx = fx_v[pl.ds(o, 16)]
            fy = fy_v[pl.ds(o, 16)]
            fz = fz_v[pl.ds(o, 16)]
            sc = scal_v[pl.ds(o, 16)]
            px = plsc.load_gather(pts_v, [obs, col[0]])
            py = plsc.load_gather(pts_v, [obs, col[1]])
            pz = plsc.load_gather(pts_v, [obs, col[2]])

            ux = mtx - sc * fx
            uy = mty - sc * fy
            uz = mtz - sc * fz
            vx, vy, vz = _rot_conj(mqx, mqy, mqz, mqw, ux, uy, uz)
            wx = vx + rtx
            wy = vy + rty
            wz = vz + rtz
            xx, xy, xz = _rot_conj(rqx, rqy, rqz, rqw, wx, wy, wz)

            ox_v[pl.ds(o, 16)] = fac * (px + xx)
            oy_v[pl.ds(o, 16)] = fac * (py + xy)
            oz_v[pl.ds(o, 16)] = fac * (pz + xz)
            return carry2

        lax.fori_loop(0, CHUNKS, chunk_body, 0)

    issue_lin(0, s0)
    drain_lin(s0)
    issue_gather(s0)
    issue_lin(1, s1)

    def body(j, carry):
        k0 = 2 * j
        k1 = 2 * j + 1

        @pl.when(k1 < nb)
        def _():
            drain_lin(s1)
            issue_gather(s1)

        @pl.when(j > 0)
        def _():
            drain_out(s0)

        drain_gather(s0)
        compute(s0)
        issue_out(k0, s0)

        @pl.when(k0 + 2 < nb)
        def _():
            issue_lin(k0 + 2, s0)

        @pl.when(k1 < nb)
        def _():
            @pl.when(j > 0)
            def _():
                drain_out(s1)

            drain_gather(s1)
            compute(s1)
            issue_out(k1, s1)

        @pl.when(k0 + 2 < nb)
        def _():
            drain_lin(s0)
            issue_gather(s0)

        @pl.when(k1 + 2 < nb)
        def _():
            issue_lin(k1 + 2, s1)

        return carry

    lax.fori_loop(0, (nb + 1) // 2, body, 0)
    drain_out(s0)

    @pl.when(nb >= 2)
    def _():
        drain_out(s1)


def _sc_call(gidx, midx, pidx, fx, fy, fz, scal, gtab, mtab, pts):
    n = gidx.shape[0]
    nblk = n // B
    mesh = plsc.VectorSubcoreMesh(core_axis_name="c", subcore_axis_name="s")
    slot = [
        pltpu.VMEM((B,), jnp.int32),
        pltpu.VMEM((B,), jnp.int32),
        pltpu.VMEM((B,), jnp.int32),
        pltpu.VMEM((B,), jnp.float32),
        pltpu.VMEM((B,), jnp.float32),
        pltpu.VMEM((B,), jnp.float32),
        pltpu.VMEM((B,), jnp.float32),
        pltpu.VMEM((B, PW), jnp.float32),
        pltpu.VMEM((B,), jnp.float32),
        pltpu.VMEM((B,), jnp.float32),
        pltpu.VMEM((B,), jnp.float32),
    ]
    import functools
    f = pl.kernel(
        functools.partial(_body, nblk),
        out_type=(jax.ShapeDtypeStruct((n,), jnp.float32),
                  jax.ShapeDtypeStruct((n,), jnp.float32),
                  jax.ShapeDtypeStruct((n,), jnp.float32)),
        mesh=mesh,
        scratch_types=[
            pltpu.VMEM((8, NUM_TAB), jnp.float32),
            pltpu.VMEM((7, NUM_TAB), jnp.float32),
        ] + slot + slot + [
            pltpu.SemaphoreType.DMA,
            pltpu.SemaphoreType.DMA,
            pltpu.SemaphoreType.DMA,
            pltpu.SemaphoreType.DMA,
            pltpu.SemaphoreType.DMA,
            pltpu.SemaphoreType.DMA,
        ],
        compiler_params=pltpu.CompilerParams(
            use_tc_tiling_on_sc=False, needs_layout_passes=False),
    )
    return f(gidx, midx, pidx, fx, fy, fz, scal, gtab, mtab, pts)


SPANS = ((0, 512000), (512000, N_OBS))


def kernel(feature_undist, grouping_indices, point_indices, is_calibrated,
           ref_rots, rel_rots, rel_trans, points_3d, scales, ref_trans):
    fac = jnp.where(is_calibrated, 1.0, 0.5).astype(jnp.float32)
    gtab = jnp.concatenate([ref_rots.T, ref_trans.T, fac[None, :]], axis=0)
    mtab = jnp.concatenate([rel_rots.T, rel_trans.T], axis=0)
    pts = jnp.pad(points_3d, ((0, 0), (0, PW - 3)))
    gi = grouping_indices.astype(jnp.int32)
    pidx = point_indices.astype(jnp.int32)
    outs = []
    for lo, hi in SPANS:
        ox, oy, oz = _sc_call(
            gi[lo:hi, 0], gi[lo:hi, 1], pidx[lo:hi],
            feature_undist[lo:hi, 0], feature_undist[lo:hi, 1],
            feature_undist[lo:hi, 2], scales[lo:hi, 0], gtab, mtab, pts)
        outs.append(jnp.stack([ox, oy, oz], axis=-1))
    return jnp.concatenate(outs, axis=0)
